# baseline (speedup 1.0000x reference)
import functools

import jax
import jax.numpy as jnp
from jax import lax
from jax.experimental import pallas as pl
from jax.experimental.pallas import tpu as pltpu
from jax.experimental.pallas import tpu_sc as plsc

EPS = 1e-6
L = 16


def _make_sc_kernel(n_nodes, d_model, n_edges):
    info = plsc.get_sparse_core_info()
    nc, ns = info.num_cores, info.num_subcores
    nw = nc * ns
    d_words = d_model // 2
    assert n_edges % nw == 0
    e_per_w = n_edges // nw
    chunk = 80
    assert e_per_w % chunk == 0
    n_chunks = e_per_w // chunk
    groups = chunk // L

    mesh = plsc.VectorSubcoreMesh(core_axis_name="c", subcore_axis_name="s")

    @functools.partial(
        pl.kernel,
        mesh=mesh,
        compiler_params=pltpu.CompilerParams(needs_layout_passes=False,
                                             use_tc_tiling_on_sc=False),
        out_type=jax.ShapeDtypeStruct((n_edges,), jnp.float32),
        scratch_types=[
            pltpu.VMEM((e_per_w,), jnp.int32),
            pltpu.VMEM((e_per_w,), jnp.int32),
            pltpu.VMEM((chunk, d_model), jnp.bfloat16),
            pltpu.VMEM((chunk, d_model), jnp.bfloat16),
            pltpu.VMEM((chunk, d_model), jnp.bfloat16),
            pltpu.VMEM((chunk, d_model), jnp.bfloat16),
            pltpu.VMEM((chunk * L,), jnp.float32),
            pltpu.VMEM((e_per_w,), jnp.float32),
            pltpu.VMEM_SHARED((n_nodes, d_model), jnp.bfloat16),
            pltpu.SemaphoreType.DMA,
            pltpu.SemaphoreType.DMA,
            pltpu.SemaphoreType.DMA,
            pltpu.SemaphoreType.DMA,
        ],
    )
    def body(z_hbm, src_hbm, dst_hbm, out_hbm,
             sidx_v, didx_v, srows0, drows0, srows1, drows1,
             psum_v, out_v, zs_sh, sem_s0, sem_d0, sem_s1, sem_d1):
        sid = lax.axis_index("s")
        wid = sid * nc + lax.axis_index("c")
        w_base = wid * e_per_w

        rows_per_tile = n_nodes // ns
        z_lo = sid * rows_per_tile
        pltpu.sync_copy(z_hbm.at[pl.ds(z_lo, rows_per_tile)],
                        zs_sh.at[pl.ds(z_lo, rows_per_tile)])

        pltpu.sync_copy(src_hbm.at[pl.ds(w_base, e_per_w)], sidx_v)
        pltpu.sync_copy(dst_hbm.at[pl.ds(w_base, e_per_w)], didx_v)
        plsc.subcore_barrier()

        bufs = ((srows0, drows0, sem_s0, sem_d0),
                (srows1, drows1, sem_s1, sem_d1))

        def fire(ci, b):
            srows, drows, sem_s, sem_d = bufs[b]
            s_sl = sidx_v.at[pl.ds(ci * chunk, chunk)]
            d_sl = didx_v.at[pl.ds(ci * chunk, chunk)]
            pltpu.async_copy(zs_sh.at[s_sl], srows, sem_s)
            pltpu.async_copy(zs_sh.at[d_sl], drows, sem_d)

        def wait(b):
            srows, drows, sem_s, sem_d = bufs[b]
            pltpu.make_async_copy(zs_sh.at[sidx_v.at[pl.ds(0, chunk)]],
                                  srows, sem_s).wait()
            pltpu.make_async_copy(zs_sh.at[didx_v.at[pl.ds(0, chunk)]],
                                  drows, sem_d).wait()

        def compute(ci, b):
            srows, drows, _, _ = bufs[b]
            lane16 = lax.iota(jnp.int32, L) * L

            def gbody(g, carry):
                pbase = lane16 + g * (L * L)
                for el0 in range(0, L, 8):
                    accq = [None] * 8
                    for u in range(d_model // (2 * L)):
                        for q in range(8):
                            e = g * L + el0 + q
                            sv = srows[e, pl.ds(u * 2 * L, 2 * L)]
                            dv = drows[e, pl.ds(u * 2 * L, 2 * L)]
                            df = sv - dv
                            p = df * df
                            accq[q] = p if accq[q] is None else accq[q] + p
                    for q in range(8):
                        pa, pb = plsc.unpack(
                            accq[q], format=plsc.PackFormat.INTERLEAVED)
                        plsc.store_scatter(psum_v, [pbase + (el0 + q)],
                                           pa + pb)
                return carry

            lax.fori_loop(0, groups, gbody, 0)

            for g in range(groups):
                acc = psum_v[pl.ds(g * L * L, L)]
                for l in range(1, L):
                    acc = acc + psum_v[pl.ds(g * L * L + l * L, L)]
                acc = acc + (d_model * EPS * EPS)
                ibits = plsc.bitcast(acc, jnp.int32)
                ibits = 0x5F3759DF - (ibits >> 1)
                y = plsc.bitcast(ibits, jnp.float32)
                y = y * (1.5 - 0.5 * acc * y * y)
                y = y * (1.5 - 0.5 * acc * y * y)
                y = y * (1.5 - 0.5 * acc * y * y)
                dist = acc * y
                out_v[pl.ds(ci * chunk + g * L, L)] = jnp.exp(-dist)

        fire(0, 0)

        def pair_body(k, carry):
            c0 = k * 4
            fire(c0 + 1, 1)
            wait(0)
            compute(c0, 0)
            fire(c0 + 2, 0)
            wait(1)
            compute(c0 + 1, 1)
            fire(c0 + 3, 1)
            wait(0)
            compute(c0 + 2, 0)
            fire(c0 + 4, 0)
            wait(1)
            compute(c0 + 3, 1)
            return carry

        lax.fori_loop(0, (n_chunks - 1) // 4, pair_body, 0)
        wait(0)
        compute(n_chunks - 1, 0)

        pltpu.sync_copy(out_v, out_hbm.at[pl.ds(w_base, e_per_w)])

    return body


def kernel(z, edge_index):
    n_nodes, d_model = z.shape
    n_edges = edge_index.shape[1]
    zb = z.astype(jnp.bfloat16)
    src = edge_index[0].astype(jnp.int32)
    dst = edge_index[1].astype(jnp.int32)
    k = _make_sc_kernel(n_nodes, d_model, n_edges)
    return k(zb, src, dst)


# speedup vs baseline: 1.1972x; 1.1972x over previous
import functools

import jax
import jax.numpy as jnp
from jax import lax
from jax.experimental import pallas as pl
from jax.experimental.pallas import tpu as pltpu
from jax.experimental.pallas import tpu_sc as plsc

EPS = 1e-6
L = 16


def _make_sc_kernel(n_nodes, d_model, n_edges):
    info = plsc.get_sparse_core_info()
    nc, ns = info.num_cores, info.num_subcores
    nw = nc * ns
    d_words = d_model // 2
    assert n_edges % nw == 0
    e_per_w = n_edges // nw
    chunk = 80
    assert e_per_w % chunk == 0
    n_chunks = e_per_w // chunk
    groups = chunk // L

    mesh = plsc.VectorSubcoreMesh(core_axis_name="c", subcore_axis_name="s")

    @functools.partial(
        pl.kernel,
        mesh=mesh,
        compiler_params=pltpu.CompilerParams(needs_layout_passes=False,
                                             use_tc_tiling_on_sc=False),
        out_type=jax.ShapeDtypeStruct((n_edges,), jnp.float32),
        scratch_types=[
            pltpu.VMEM((e_per_w,), jnp.int32),
            pltpu.VMEM((e_per_w,), jnp.int32),
            pltpu.VMEM((chunk, d_model), jnp.bfloat16),
            pltpu.VMEM((chunk, d_model), jnp.bfloat16),
            pltpu.VMEM((chunk, d_model), jnp.bfloat16),
            pltpu.VMEM((chunk, d_model), jnp.bfloat16),
            pltpu.VMEM((chunk * L,), jnp.int32),
            pltpu.VMEM((e_per_w,), jnp.float32),
            pltpu.VMEM_SHARED((n_nodes, d_model), jnp.bfloat16),
            pltpu.SemaphoreType.DMA,
            pltpu.SemaphoreType.DMA,
            pltpu.SemaphoreType.DMA,
            pltpu.SemaphoreType.DMA,
        ],
    )
    def body(z_hbm, src_hbm, dst_hbm, out_hbm,
             sidx_v, didx_v, srows0, drows0, srows1, drows1,
             psum_i, out_v, zs_sh, sem_s0, sem_d0, sem_s1, sem_d1):
        sid = lax.axis_index("s")
        wid = sid * nc + lax.axis_index("c")
        w_base = wid * e_per_w

        rows_per_tile = n_nodes // ns
        z_lo = sid * rows_per_tile
        pltpu.sync_copy(z_hbm.at[pl.ds(z_lo, rows_per_tile)],
                        zs_sh.at[pl.ds(z_lo, rows_per_tile)])

        pltpu.sync_copy(src_hbm.at[pl.ds(w_base, e_per_w)], sidx_v)
        pltpu.sync_copy(dst_hbm.at[pl.ds(w_base, e_per_w)], didx_v)
        plsc.subcore_barrier()

        bufs = ((srows0, drows0, sem_s0, sem_d0),
                (srows1, drows1, sem_s1, sem_d1))

        def fire(ci, b):
            srows, drows, sem_s, sem_d = bufs[b]
            s_sl = sidx_v.at[pl.ds(ci * chunk, chunk)]
            d_sl = didx_v.at[pl.ds(ci * chunk, chunk)]
            pltpu.async_copy(zs_sh.at[s_sl], srows, sem_s)
            pltpu.async_copy(zs_sh.at[d_sl], drows, sem_d)

        def wait(b):
            srows, drows, sem_s, sem_d = bufs[b]
            pltpu.make_async_copy(zs_sh.at[sidx_v.at[pl.ds(0, chunk)]],
                                  srows, sem_s).wait()
            pltpu.make_async_copy(zs_sh.at[didx_v.at[pl.ds(0, chunk)]],
                                  drows, sem_d).wait()

        def compute(ci, b):
            srows, drows, _, _ = bufs[b]
            lane16 = lax.iota(jnp.int32, L) * L

            def gbody(g, carry):
                pbase = lane16 + g * (L * L)
                for el0 in range(0, L, 8):
                    accq = [None] * 8
                    for u in range(d_model // (2 * L)):
                        for q in range(8):
                            e = g * L + el0 + q
                            sv = srows[e, pl.ds(u * 2 * L, 2 * L)]
                            dv = drows[e, pl.ds(u * 2 * L, 2 * L)]
                            df = sv - dv
                            p = df * df
                            accq[q] = p if accq[q] is None else accq[q] + p
                    for q in range(8):
                        plsc.store_scatter(
                            psum_i, [pbase + (el0 + q)],
                            plsc.bitcast(accq[q], jnp.int32))
                return carry

            lax.fori_loop(0, groups, gbody, 0)

            for g in range(groups):
                accb = plsc.bitcast(psum_i[pl.ds(g * L * L, L)],
                                    jnp.bfloat16)
                for l in range(1, L):
                    accb = accb + plsc.bitcast(
                        psum_i[pl.ds(g * L * L + l * L, L)], jnp.bfloat16)
                pa, pb = plsc.unpack(accb,
                                     format=plsc.PackFormat.INTERLEAVED)
                acc = pa + pb
                acc = acc + (d_model * EPS * EPS)
                ibits = plsc.bitcast(acc, jnp.int32)
                ibits = 0x5F3759DF - (ibits >> 1)
                y = plsc.bitcast(ibits, jnp.float32)
                y = y * (1.5 - 0.5 * acc * y * y)
                y = y * (1.5 - 0.5 * acc * y * y)
                dist = acc * y
                out_v[pl.ds(ci * chunk + g * L, L)] = jnp.exp(-dist)

        fire(0, 0)

        def pair_body(k, carry):
            c0 = k * 2
            fire(c0 + 1, 1)
            wait(0)
            compute(c0, 0)
            fire(c0 + 2, 0)
            wait(1)
            compute(c0 + 1, 1)
            return carry

        lax.fori_loop(0, (n_chunks - 1) // 2, pair_body, 0)
        wait(0)
        compute(n_chunks - 1, 0)

        pltpu.sync_copy(out_v, out_hbm.at[pl.ds(w_base, e_per_w)])

    return body


def kernel(z, edge_index):
    n_nodes, d_model = z.shape
    n_edges = edge_index.shape[1]
    zb = z.astype(jnp.bfloat16)
    src = edge_index[0].astype(jnp.int32)
    dst = edge_index[1].astype(jnp.int32)
    k = _make_sc_kernel(n_nodes, d_model, n_edges)
    return k(zb, src, dst)


# speedup vs baseline: 1.2251x; 1.0233x over previous
import functools

import jax
import jax.numpy as jnp
from jax import lax
from jax.experimental import pallas as pl
from jax.experimental.pallas import tpu as pltpu
from jax.experimental.pallas import tpu_sc as plsc

EPS = 1e-6
L = 16


def _make_sc_kernel(n_nodes, d_model, n_edges):
    info = plsc.get_sparse_core_info()
    nc, ns = info.num_cores, info.num_subcores
    nw = nc * ns
    d_words = d_model // 2
    assert n_edges % nw == 0
    e_per_w = n_edges // nw
    chunk = 80
    assert e_per_w % chunk == 0
    n_chunks = e_per_w // chunk
    groups = chunk // L

    mesh = plsc.VectorSubcoreMesh(core_axis_name="c", subcore_axis_name="s")

    @functools.partial(
        pl.kernel,
        mesh=mesh,
        compiler_params=pltpu.CompilerParams(needs_layout_passes=False,
                                             use_tc_tiling_on_sc=False),
        out_type=jax.ShapeDtypeStruct((n_edges,), jnp.float32),
        scratch_types=[
            pltpu.VMEM((e_per_w,), jnp.int32),
            pltpu.VMEM((e_per_w,), jnp.int32),
            pltpu.VMEM((chunk, d_model), jnp.bfloat16),
            pltpu.VMEM((chunk, d_model), jnp.bfloat16),
            pltpu.VMEM((chunk, d_model), jnp.bfloat16),
            pltpu.VMEM((chunk, d_model), jnp.bfloat16),
            pltpu.VMEM((chunk * L,), jnp.int32),
            pltpu.VMEM((e_per_w,), jnp.float32),
            pltpu.VMEM_SHARED((n_nodes, d_model), jnp.bfloat16),
            pltpu.SemaphoreType.DMA,
            pltpu.SemaphoreType.DMA,
            pltpu.SemaphoreType.DMA,
            pltpu.SemaphoreType.DMA,
        ],
    )
    def body(z_hbm, src_hbm, dst_hbm, out_hbm,
             sidx_v, didx_v, srows0, drows0, srows1, drows1,
             psum_i, out_v, zs_sh, sem_s0, sem_d0, sem_s1, sem_d1):
        sid = lax.axis_index("s")
        wid = sid * nc + lax.axis_index("c")
        w_base = wid * e_per_w

        rows_per_tile = n_nodes // ns
        z_lo = sid * rows_per_tile
        pltpu.sync_copy(z_hbm.at[pl.ds(z_lo, rows_per_tile)],
                        zs_sh.at[pl.ds(z_lo, rows_per_tile)])

        pltpu.sync_copy(src_hbm.at[pl.ds(w_base, e_per_w)], sidx_v)
        pltpu.sync_copy(dst_hbm.at[pl.ds(w_base, e_per_w)], didx_v)
        plsc.subcore_barrier()

        bufs = ((srows0, drows0, sem_s0, sem_d0),
                (srows1, drows1, sem_s1, sem_d1))

        def fire(ci, b):
            srows, drows, sem_s, sem_d = bufs[b]
            s_sl = sidx_v.at[pl.ds(ci * chunk, chunk)]
            d_sl = didx_v.at[pl.ds(ci * chunk, chunk)]
            pltpu.async_copy(zs_sh.at[s_sl], srows, sem_s)
            pltpu.async_copy(zs_sh.at[d_sl], drows, sem_d)

        def wait(b):
            srows, drows, sem_s, sem_d = bufs[b]
            pltpu.make_async_copy(zs_sh.at[sidx_v.at[pl.ds(0, chunk)]],
                                  srows, sem_s).wait()
            pltpu.make_async_copy(zs_sh.at[didx_v.at[pl.ds(0, chunk)]],
                                  drows, sem_d).wait()

        def compute(ci, b):
            srows, drows, _, _ = bufs[b]
            lane16 = lax.iota(jnp.int32, L) * L

            def gbody(g, carry):
                pbase = lane16 + g * (L * L)
                for el0 in range(0, L, 8):
                    accq = [None] * 8
                    for u in range(d_model // (2 * L)):
                        for q in range(8):
                            e = g * L + el0 + q
                            sv = srows[e, pl.ds(u * 2 * L, 2 * L)]
                            dv = drows[e, pl.ds(u * 2 * L, 2 * L)]
                            df = sv - dv
                            p = df * df
                            accq[q] = p if accq[q] is None else accq[q] + p
                    for q in range(8):
                        plsc.store_scatter(
                            psum_i, [pbase + (el0 + q)],
                            plsc.bitcast(accq[q], jnp.int32))
                return carry

            lax.fori_loop(0, groups, gbody, 0)

            for g in range(groups):
                vs = [plsc.bitcast(psum_i[pl.ds(g * L * L + l * L, L)],
                                   jnp.bfloat16) for l in range(L)]
                while len(vs) > 1:
                    vs = [a + b for a, b in zip(vs[::2], vs[1::2])]
                accb = vs[0]
                pa, pb = plsc.unpack(accb,
                                     format=plsc.PackFormat.INTERLEAVED)
                acc = pa + pb
                acc = acc + (d_model * EPS * EPS)
                ibits = plsc.bitcast(acc, jnp.int32)
                ibits = 0x5F3759DF - (ibits >> 1)
                y = plsc.bitcast(ibits, jnp.float32)
                y = y * (1.5 - 0.5 * acc * y * y)
                y = y * (1.5 - 0.5 * acc * y * y)
                dist = acc * y
                out_v[pl.ds(ci * chunk + g * L, L)] = jnp.exp(-dist)

        fire(0, 0)

        def pair_body(k, carry):
            c0 = k * 2
            fire(c0 + 1, 1)
            wait(0)
            compute(c0, 0)
            fire(c0 + 2, 0)
            wait(1)
            compute(c0 + 1, 1)
            return carry

        lax.fori_loop(0, (n_chunks - 1) // 2, pair_body, 0)
        wait(0)
        compute(n_chunks - 1, 0)

        pltpu.sync_copy(out_v, out_hbm.at[pl.ds(w_base, e_per_w)])

    return body


def kernel(z, edge_index):
    n_nodes, d_model = z.shape
    n_edges = edge_index.shape[1]
    zb = z.astype(jnp.bfloat16)
    src = edge_index[0].astype(jnp.int32)
    dst = edge_index[1].astype(jnp.int32)
    k = _make_sc_kernel(n_nodes, d_model, n_edges)
    return k(zb, src, dst)
